# baseline (speedup 1.0000x reference)
import functools

import jax
import jax.numpy as jnp
from jax import lax
from jax.experimental import pallas as pl
from jax.experimental.pallas import tpu as pltpu
from jax.experimental.pallas import tpu_sc as plsc

B, S = 64, 8192
L = 16
NCHUNK = S // L
NW = 32
ROWS_PER_W = B // NW
RANK = S // 2 + 1
NBIN = 1024
HCHUNK = NBIN // L
ACC = 4

_mesh = plsc.VectorSubcoreMesh(core_axis_name="c", subcore_axis_name="s")


def _bcast(x, lane):
    return x.at[lane].get(mode="promise_in_bounds")


def _butterfly_max(x):
    lanes = lax.iota(jnp.int32, L)
    for s in (1, 2, 4, 8):
        x = jnp.maximum(x, x.at[lanes ^ s].get(mode="promise_in_bounds"))
    return x


@functools.partial(
    pl.kernel,
    out_type=jax.ShapeDtypeStruct((B, S), jnp.float32),
    mesh=_mesh,
    compiler_params=pltpu.CompilerParams(needs_layout_passes=False),
    scratch_types=[
        pltpu.VMEM((S,), jnp.float32),
        pltpu.VMEM((S,), jnp.float32),
        pltpu.VMEM((S,), jnp.float32),
        pltpu.VMEM((S,), jnp.float32),
        pltpu.VMEM((S,), jnp.int32),
        pltpu.VMEM((NBIN,), jnp.int32),
        pltpu.VMEM((NBIN,), jnp.int32),
        pltpu.VMEM((L,), jnp.float32),
        pltpu.VMEM((L,), jnp.float32),
        pltpu.SemaphoreType.DMA,
        pltpu.SemaphoreType.DMA,
        pltpu.SemaphoreType.DMA,
        pltpu.SemaphoreType.DMA,
        pltpu.SemaphoreType.DMA,
        pltpu.SemaphoreType.DMA,
        pltpu.SemaphoreType.DMA,
    ],
)
def _sparse_importance_sc(attn_hbm, grad_hbm, wa_hbm, wg_hbm, out_hbm,
                          a0_v, a1_v, g0_v, g1_v, cb_v, hist_v, csum_v,
                          wa_v, wg_v, sa0, sg0, sa1, sg1, so0, so1, sw):
    wid = lax.axis_index("s") * 2 + lax.axis_index("c")
    row0 = wid * ROWS_PER_W

    bufs = ((a0_v, g0_v, sa0, sg0), (a1_v, g1_v, sa1, sg1))
    in_cps = []
    for r, (ab, gb, sa, sg) in enumerate(bufs):
        in_cps.append((
            pltpu.async_copy(attn_hbm.at[row0 + r], ab, sa),
            pltpu.async_copy(grad_hbm.at[row0 + r], gb, sg)))
    w_cps = (pltpu.async_copy(wa_hbm, wa_v, sw),
             pltpu.async_copy(wg_hbm, wg_v, sw))

    zero_iv = jnp.zeros((L,), jnp.int32)
    one_iv = jnp.full((L,), 1, jnp.int32)
    rank_v = jnp.full((L,), RANK, jnp.int32)
    mask1023_v = jnp.full((L,), NBIN - 1, jnp.int32)
    lane15_v = jnp.full((L,), L - 1, jnp.int32)
    zero_fv = jnp.zeros((L,), jnp.float32)

    def zero_hist(i):
        hist_v[pl.ds(i * L, L)] = zero_iv

    lanes_v = lax.iota(jnp.int32, L)

    def hist_scan(rank_need_v):
        def csum_body(i):
            h = hist_v[pl.ds(i * L, L)]
            hist_v[pl.ds(i * L, L)] = zero_iv
            csum_v[pl.ds(i * L, L)] = plsc.cumsum(h)
        plsc.parallel_loop(0, HCHUNK, unroll=2)(csum_body)

        prev_v = zero_iv
        j0_v = jnp.full((L,), -1, jnp.int32)
        below_ch_v = zero_iv
        for t in range(HCHUNK // L):
            idx = (jnp.full((L,), t * L, jnp.int32) + lanes_v) * L + (L - 1)
            tots = plsc.load_gather(csum_v, [idx])
            cs2 = prev_v + plsc.cumsum(tots)
            cross = cs2 >= rank_need_v
            pc = plsc.all_reduce_population_count(cross)
            ff = plsc.all_reduce_ffs(cross)
            ffc = jnp.minimum(ff, lane15_v)
            hit = jnp.where(j0_v < 0,
                            jnp.where(pc > 0, one_iv, zero_iv), zero_iv)
            base_v = jnp.full((L,), t * L, jnp.int32)
            j0_v = jnp.where(hit > 0, base_v + ffc, j0_v)
            cs2_at = _bcast(cs2, ffc)
            t_at = _bcast(tots, ffc)
            below_ch_v = jnp.where(hit > 0, cs2_at - t_at, below_ch_v)
            prev_v = _bcast(cs2, lane15_v)

        j0 = jnp.max(j0_v)
        cs0 = csum_v[pl.ds(j0 * L, L)]
        cst = below_ch_v + cs0
        cross = cst >= rank_need_v
        ffc = jnp.minimum(plsc.all_reduce_ffs(cross), lane15_v)
        b_v = j0_v * L + ffc
        ffm1 = jnp.maximum(ffc - 1, zero_iv)
        below_in = jnp.where(ffc > 0, _bcast(cs0, ffm1), zero_iv)
        below_v = below_ch_v + below_in
        return b_v, below_v

    out_cps = []
    for r in range(ROWS_PER_W):
        row = row0 + r
        for cp in in_cps[r]:
            cp.wait()
        ar, gr = bufs[r][0], bufs[r][1]

        def max_body(i, carry):
            new = []
            for k in range(ACC):
                am, gm = carry[k]
                a = ar[pl.ds((i * ACC + k) * L, L)]
                g = gr[pl.ds((i * ACC + k) * L, L)]
                new.append((jnp.maximum(am, a), jnp.maximum(gm, g)))
            return tuple(new)
        neg_fv = jnp.full((L,), -1.0, jnp.float32)
        accs = plsc.parallel_loop(
            0, NCHUNK // ACC, unroll=1,
            carry=tuple((neg_fv, neg_fv) for _ in range(ACC)))(max_body)
        am_v, gm_v = accs[0]
        for k in range(1, ACC):
            am_v = jnp.maximum(am_v, accs[k][0])
            gm_v = jnp.maximum(gm_v, accs[k][1])
        amax_v = _butterfly_max(am_v)
        gmax_v = _butterfly_max(gm_v)
        if r == 0:
            for cp in w_cps:
                cp.wait()
        ca_v = wa_v[...] / (amax_v + 1e-8)
        cg_v = wg_v[...] / (gmax_v + 1e-8)

        if r == 0:
            plsc.parallel_loop(0, HCHUNK, unroll=4)(zero_hist)

        def comb_body(i, carry):
            new = []
            for k in range(ACC):
                cmaxb = carry[k]
                a = ar[pl.ds((i * ACC + k) * L, L)]
                g = gr[pl.ds((i * ACC + k) * L, L)]
                c = ca_v * a + cg_v * g
                cb = lax.bitcast_convert_type(c, jnp.int32)
                cb_v[pl.ds((i * ACC + k) * L, L)] = cb
                bin1 = jnp.minimum(lax.shift_right_logical(cb, 20), mask1023_v)
                plsc.addupdate_scatter(hist_v, [bin1], one_iv)
                new.append(jnp.maximum(cmaxb, cb))
            return tuple(new)
        baccs = plsc.parallel_loop(
            0, NCHUNK // ACC, unroll=1,
            carry=tuple(zero_iv for _ in range(ACC)))(comb_body)
        cmaxb_v = baccs[0]
        for k in range(1, ACC):
            cmaxb_v = jnp.maximum(cmaxb_v, baccs[k])
        cmax_bv = _butterfly_max(cmaxb_v)

        b0_v, below0_v = hist_scan(rank_v)
        rank2_v = rank_v - below0_v

        def h2_body(i):
            for k in range(ACC):
                cb = cb_v[pl.ds((i * ACC + k) * L, L)]
                top = lax.shift_right_logical(cb, 20)
                bin2 = lax.shift_right_logical(cb, 10) & mask1023_v
                plsc.addupdate_scatter(hist_v, [bin2], one_iv,
                                       mask=top == b0_v)
        plsc.parallel_loop(0, NCHUNK // ACC, unroll=1)(h2_body)
        b1_v, below1_v = hist_scan(rank2_v)
        rank3_v = rank2_v - below1_v
        top20_v = lax.shift_left(b0_v, 10) | b1_v

        def h3_body(i):
            for k in range(ACC):
                cb = cb_v[pl.ds((i * ACC + k) * L, L)]
                hi20 = lax.shift_right_logical(cb, 10)
                bin3 = cb & mask1023_v
                plsc.addupdate_scatter(hist_v, [bin3], one_iv,
                                       mask=hi20 == top20_v)
        plsc.parallel_loop(0, NCHUNK // ACC, unroll=1)(h3_body)
        b2_v, _ = hist_scan(rank3_v)
        thr_v = lax.shift_left(top20_v, 10) | b2_v

        cmax_f = lax.bitcast_convert_type(cmax_bv, jnp.float32)
        inv_v = jnp.full((L,), 1.0, jnp.float32) / (cmax_f + 1e-8)

        def out_body(i):
            for k in range(ACC):
                cb = cb_v[pl.ds((i * ACC + k) * L, L)]
                c = lax.bitcast_convert_type(cb, jnp.float32)
                o = jnp.where(cb > thr_v, c * inv_v, zero_fv)
                ar[pl.ds((i * ACC + k) * L, L)] = o
        plsc.parallel_loop(0, NCHUNK // ACC, unroll=1)(out_body)
        out_cps.append(pltpu.async_copy(ar, out_hbm.at[row],
                                        (so0, so1)[r]))
    for cp in out_cps:
        cp.wait()


def kernel(attention_weights, gradient_importance, attention_weight,
           gradient_weight):
    w = jax.nn.softmax(jnp.stack([attention_weight, gradient_weight]), axis=0)
    wa = jnp.full((L,), w[0], jnp.float32)
    wg = jnp.full((L,), w[1], jnp.float32)
    return _sparse_importance_sc(attention_weights, gradient_importance,
                                 wa, wg)


# speedup vs baseline: 1.0060x; 1.0060x over previous
import functools

import jax
import jax.numpy as jnp
from jax import lax
from jax.experimental import pallas as pl
from jax.experimental.pallas import tpu as pltpu
from jax.experimental.pallas import tpu_sc as plsc

B, S = 64, 8192
L = 16
NCHUNK = S // L
NW = 32
ROWS_PER_W = B // NW
RANK = S // 2 + 1
NBIN = 1024
HCHUNK = NBIN // L
ACC = 4

_mesh = plsc.VectorSubcoreMesh(core_axis_name="c", subcore_axis_name="s")


def _bcast(x, lane):
    return x.at[lane].get(mode="promise_in_bounds")


def _butterfly_max(x):
    lanes = lax.iota(jnp.int32, L)
    for s in (1, 2, 4, 8):
        x = jnp.maximum(x, x.at[lanes ^ s].get(mode="promise_in_bounds"))
    return x


@functools.partial(
    pl.kernel,
    out_type=jax.ShapeDtypeStruct((B, S), jnp.float32),
    mesh=_mesh,
    compiler_params=pltpu.CompilerParams(needs_layout_passes=False),
    scratch_types=[
        pltpu.VMEM((S,), jnp.float32),
        pltpu.VMEM((S,), jnp.float32),
        pltpu.VMEM((S,), jnp.float32),
        pltpu.VMEM((S,), jnp.float32),
        pltpu.VMEM((S,), jnp.int32),
        pltpu.VMEM((NBIN,), jnp.int32),
        pltpu.VMEM((NBIN,), jnp.int32),
        pltpu.VMEM((L,), jnp.float32),
        pltpu.VMEM((L,), jnp.float32),
        pltpu.SemaphoreType.DMA,
        pltpu.SemaphoreType.DMA,
        pltpu.SemaphoreType.DMA,
        pltpu.SemaphoreType.DMA,
        pltpu.SemaphoreType.DMA,
        pltpu.SemaphoreType.DMA,
        pltpu.SemaphoreType.DMA,
    ],
)
def _sparse_importance_sc(attn_hbm, grad_hbm, wa_hbm, wg_hbm, out_hbm,
                          a0_v, a1_v, g0_v, g1_v, cb_v, hist_v, csum_v,
                          wa_v, wg_v, sa0, sg0, sa1, sg1, so0, so1, sw):
    wid = lax.axis_index("s") * 2 + lax.axis_index("c")
    row0 = wid * ROWS_PER_W

    bufs = ((a0_v, g0_v, sa0, sg0), (a1_v, g1_v, sa1, sg1))
    in_cps = []
    for r, (ab, gb, sa, sg) in enumerate(bufs):
        in_cps.append((
            pltpu.async_copy(attn_hbm.at[row0 + r], ab, sa),
            pltpu.async_copy(grad_hbm.at[row0 + r], gb, sg)))
    w_cps = (pltpu.async_copy(wa_hbm, wa_v, sw),
             pltpu.async_copy(wg_hbm, wg_v, sw))

    zero_iv = jnp.zeros((L,), jnp.int32)
    one_iv = jnp.full((L,), 1, jnp.int32)
    rank_v = jnp.full((L,), RANK, jnp.int32)
    mask1023_v = jnp.full((L,), NBIN - 1, jnp.int32)
    lane15_v = jnp.full((L,), L - 1, jnp.int32)
    zero_fv = jnp.zeros((L,), jnp.float32)

    def zero_hist(i):
        hist_v[pl.ds(i * L, L)] = zero_iv

    lanes_v = lax.iota(jnp.int32, L)

    def hist_scan(rank_need_v):
        def csum_body(i):
            h = hist_v[pl.ds(i * L, L)]
            hist_v[pl.ds(i * L, L)] = zero_iv
            csum_v[pl.ds(i * L, L)] = plsc.cumsum(h)
        plsc.parallel_loop(0, HCHUNK, unroll=2)(csum_body)

        prev_v = zero_iv
        j0_v = jnp.full((L,), -1, jnp.int32)
        below_ch_v = zero_iv
        for t in range(HCHUNK // L):
            idx = (jnp.full((L,), t * L, jnp.int32) + lanes_v) * L + (L - 1)
            tots = plsc.load_gather(csum_v, [idx])
            cs2 = prev_v + plsc.cumsum(tots)
            cross = cs2 >= rank_need_v
            pc = plsc.all_reduce_population_count(cross)
            ff = plsc.all_reduce_ffs(cross)
            ffc = jnp.minimum(ff, lane15_v)
            hit = jnp.where(j0_v < 0,
                            jnp.where(pc > 0, one_iv, zero_iv), zero_iv)
            base_v = jnp.full((L,), t * L, jnp.int32)
            j0_v = jnp.where(hit > 0, base_v + ffc, j0_v)
            cs2_at = _bcast(cs2, ffc)
            t_at = _bcast(tots, ffc)
            below_ch_v = jnp.where(hit > 0, cs2_at - t_at, below_ch_v)
            prev_v = _bcast(cs2, lane15_v)

        j0 = jnp.max(j0_v)
        cs0 = csum_v[pl.ds(j0 * L, L)]
        cst = below_ch_v + cs0
        cross = cst >= rank_need_v
        ffc = jnp.minimum(plsc.all_reduce_ffs(cross), lane15_v)
        b_v = j0_v * L + ffc
        ffm1 = jnp.maximum(ffc - 1, zero_iv)
        below_in = jnp.where(ffc > 0, _bcast(cs0, ffm1), zero_iv)
        below_v = below_ch_v + below_in
        return b_v, below_v

    out_cps = []
    for r in range(ROWS_PER_W):
        row = row0 + r
        for cp in in_cps[r]:
            cp.wait()
        ar, gr = bufs[r][0], bufs[r][1]

        def max_body(i, carry):
            new = []
            for k in range(ACC):
                am, gm = carry[k]
                a = ar[pl.ds((i * ACC + k) * L, L)]
                g = gr[pl.ds((i * ACC + k) * L, L)]
                new.append((jnp.maximum(am, a), jnp.maximum(gm, g)))
            return tuple(new)
        neg_fv = jnp.full((L,), -1.0, jnp.float32)
        accs = plsc.parallel_loop(
            0, NCHUNK // ACC, unroll=2,
            carry=tuple((neg_fv, neg_fv) for _ in range(ACC)))(max_body)
        am_v, gm_v = accs[0]
        for k in range(1, ACC):
            am_v = jnp.maximum(am_v, accs[k][0])
            gm_v = jnp.maximum(gm_v, accs[k][1])
        amax_v = _butterfly_max(am_v)
        gmax_v = _butterfly_max(gm_v)
        if r == 0:
            for cp in w_cps:
                cp.wait()
        ca_v = wa_v[...] / (amax_v + 1e-8)
        cg_v = wg_v[...] / (gmax_v + 1e-8)

        if r == 0:
            plsc.parallel_loop(0, HCHUNK, unroll=4)(zero_hist)

        def comb_body(i, carry):
            new = []
            for k in range(ACC):
                cmaxb = carry[k]
                a = ar[pl.ds((i * ACC + k) * L, L)]
                g = gr[pl.ds((i * ACC + k) * L, L)]
                c = ca_v * a + cg_v * g
                cb = lax.bitcast_convert_type(c, jnp.int32)
                cb_v[pl.ds((i * ACC + k) * L, L)] = cb
                bin1 = jnp.minimum(lax.shift_right_logical(cb, 20), mask1023_v)
                plsc.addupdate_scatter(hist_v, [bin1], one_iv)
                new.append(jnp.maximum(cmaxb, cb))
            return tuple(new)
        baccs = plsc.parallel_loop(
            0, NCHUNK // ACC, unroll=2,
            carry=tuple(zero_iv for _ in range(ACC)))(comb_body)
        cmaxb_v = baccs[0]
        for k in range(1, ACC):
            cmaxb_v = jnp.maximum(cmaxb_v, baccs[k])
        cmax_bv = _butterfly_max(cmaxb_v)

        b0_v, below0_v = hist_scan(rank_v)
        rank2_v = rank_v - below0_v

        def h2_body(i):
            for k in range(ACC):
                cb = cb_v[pl.ds((i * ACC + k) * L, L)]
                top = lax.shift_right_logical(cb, 20)
                bin2 = lax.shift_right_logical(cb, 10) & mask1023_v
                plsc.addupdate_scatter(hist_v, [bin2], one_iv,
                                       mask=top == b0_v)
        plsc.parallel_loop(0, NCHUNK // ACC, unroll=2)(h2_body)
        b1_v, below1_v = hist_scan(rank2_v)
        rank3_v = rank2_v - below1_v
        top20_v = lax.shift_left(b0_v, 10) | b1_v

        def h3_body(i):
            for k in range(ACC):
                cb = cb_v[pl.ds((i * ACC + k) * L, L)]
                hi20 = lax.shift_right_logical(cb, 10)
                bin3 = cb & mask1023_v
                plsc.addupdate_scatter(hist_v, [bin3], one_iv,
                                       mask=hi20 == top20_v)
        plsc.parallel_loop(0, NCHUNK // ACC, unroll=2)(h3_body)
        b2_v, _ = hist_scan(rank3_v)
        thr_v = lax.shift_left(top20_v, 10) | b2_v

        cmax_f = lax.bitcast_convert_type(cmax_bv, jnp.float32)
        inv_v = jnp.full((L,), 1.0, jnp.float32) / (cmax_f + 1e-8)

        def out_body(i):
            for k in range(ACC):
                cb = cb_v[pl.ds((i * ACC + k) * L, L)]
                c = lax.bitcast_convert_type(cb, jnp.float32)
                o = jnp.where(cb > thr_v, c * inv_v, zero_fv)
                ar[pl.ds((i * ACC + k) * L, L)] = o
        plsc.parallel_loop(0, NCHUNK // ACC, unroll=2)(out_body)
        out_cps.append(pltpu.async_copy(ar, out_hbm.at[row],
                                        (so0, so1)[r]))
    for cp in out_cps:
        cp.wait()


def kernel(attention_weights, gradient_importance, attention_weight,
           gradient_weight):
    w = jax.nn.softmax(jnp.stack([attention_weight, gradient_weight]), axis=0)
    wa = jnp.full((L,), w[0], jnp.float32)
    wg = jnp.full((L,), w[1], jnp.float32)
    return _sparse_importance_sc(attention_weights, gradient_importance,
                                 wa, wg)
